# Initial kernel scaffold; baseline (speedup 1.0000x reference)
import jax
import jax.numpy as jnp
from jax.experimental import pallas as pl


def kernel(train_feature, train_label):
    raise NotImplementedError("write your pallas kernel here")


# speedup vs baseline: 15.7106x; 15.7106x over previous
import functools

import jax
import jax.numpy as jnp
from jax.experimental import pallas as pl
from jax.experimental.pallas import tpu as pltpu

K = 16
ROWS = 256
MASKED = 1e30


def _knn_body(src_ref, all_ref, lab_ref, alllab_ref, sqs_ref, sqd_ref, out_ref):
    src = src_ref[0]
    allf = all_ref[0]
    lab = lab_ref[0]
    alllab = alllab_ref[0]

    dn = (((1,), (1,)), ((), ()))
    dots = jax.lax.dot_general(src, allf, dn)
    same = jax.lax.dot_general(lab, alllab, dn)

    sq_s = sqs_ref[0, 0]
    sq_d = sqd_ref[0, 0]
    dist = sq_s[:, None] - 2.0 * dots + sq_d[None, :]
    dist = jnp.maximum(dist, 0.0)

    bits = jax.lax.bitcast_convert_type(dist, jnp.uint32)
    keys = jax.lax.bitcast_convert_type(
        (bits & jnp.uint32(0xFFFFFFFE)) | same.astype(jnp.uint32),
        jnp.float32)

    num = jnp.zeros((ROWS,), jnp.float32)
    den = jnp.zeros((ROWS,), jnp.float32)
    cnt = jnp.zeros((ROWS,), jnp.float32)
    for _ in range(K):
        m = jnp.min(keys, axis=1)
        eq = keys == m[:, None]
        n_eq = jnp.sum(eq.astype(jnp.float32), axis=1)
        w = jnp.clip(jnp.float32(K) - cnt, 0.0, n_eq)
        mbit = jax.lax.bitcast_convert_type(m, jnp.uint32) & jnp.uint32(1)
        is_same = mbit.astype(jnp.float32)
        contrib = jnp.where(w > 0.0, w * m, 0.0)
        num = num + contrib * (1.0 - is_same)
        den = den + contrib * is_same
        cnt = cnt + n_eq
        keys = jnp.where(eq, jnp.float32(MASKED), keys)

    out_ref[0, 0, 0, :] = num / (den + 1e-8)


@functools.partial(jax.jit, static_argnames=("interpret",))
def kernel(train_feature, train_label, interpret=False):
    B, N, C = train_feature.shape
    L = train_label.shape[-1]
    sq = jnp.sum(train_feature * train_feature, axis=-1)
    sq3 = sq[:, None, :]
    grid = (B, N // ROWS)
    return pl.pallas_call(
        _knn_body,
        grid=grid,
        in_specs=[
            pl.BlockSpec((1, ROWS, C), lambda b, i: (b, i, 0)),
            pl.BlockSpec((1, N, C), lambda b, i: (b, 0, 0)),
            pl.BlockSpec((1, ROWS, L), lambda b, i: (b, i, 0)),
            pl.BlockSpec((1, N, L), lambda b, i: (b, 0, 0)),
            pl.BlockSpec((1, 1, ROWS), lambda b, i: (b, 0, i)),
            pl.BlockSpec((1, 1, N), lambda b, i: (b, 0, 0)),
        ],
        out_specs=pl.BlockSpec((1, 1, 1, ROWS), lambda b, i: (b, i, 0, 0)),
        out_shape=jax.ShapeDtypeStruct((B, N // ROWS, 1, ROWS), jnp.float32),
        interpret=interpret,
    )(train_feature, train_feature, train_label, train_label, sq3, sq3
      ).reshape(B, N)


# speedup vs baseline: 26.6157x; 1.6941x over previous
import functools

import jax
import jax.numpy as jnp
from jax.experimental import pallas as pl
from jax.experimental.pallas import tpu as pltpu

K = 16
ROWS = 256
MASKED = 1e30
SLABS = 16


def _oddeven_merge(lo, hi, r, pairs):
    step = r * 2
    if step < hi - lo:
        _oddeven_merge(lo, hi, step, pairs)
        _oddeven_merge(lo + r, hi, step, pairs)
        for i in range(lo + r, hi - r, step):
            pairs.append((i, i + r))
    else:
        pairs.append((lo, lo + r))


def _oddeven_merge_sort(lo, hi, pairs):
    if hi - lo >= 1:
        mid = lo + (hi - lo) // 2
        _oddeven_merge_sort(lo, mid, pairs)
        _oddeven_merge_sort(mid + 1, hi, pairs)
        _oddeven_merge(lo, hi, 1, pairs)


_SORT_PAIRS = []
_oddeven_merge_sort(0, SLABS - 1, _SORT_PAIRS)


def _knn_body(src_ref, all_ref, lab_ref, alllab_ref, sqs_ref, sqd_ref, out_ref):
    src = src_ref[0]
    allf = all_ref[0]
    lab = lab_ref[0]
    alllab = alllab_ref[0]

    dn = (((1,), (1,)), ((), ()))
    dots = jax.lax.dot_general(src, allf, dn)
    same = jax.lax.dot_general(lab, alllab, dn)

    sq_s = sqs_ref[0, 0]
    sq_d = sqd_ref[0, 0]
    dist = sq_s[:, None] - 2.0 * dots + sq_d[None, :]
    dist = jnp.maximum(dist, 0.0)

    bits = jax.lax.bitcast_convert_type(dist, jnp.uint32)
    keys = jax.lax.bitcast_convert_type(
        (bits & jnp.uint32(0xFFFFFFFE)) | same.astype(jnp.uint32),
        jnp.float32)

    N = keys.shape[1]
    W = N // SLABS
    slabs = [keys[:, s * W:(s + 1) * W] for s in range(SLABS)]
    for i, j in _SORT_PAIRS:
        a, b = slabs[i], slabs[j]
        slabs[i] = jnp.minimum(a, b)
        slabs[j] = jnp.maximum(a, b)

    num = jnp.zeros((ROWS,), jnp.float32)
    den = jnp.zeros((ROWS,), jnp.float32)
    cnt = jnp.zeros((ROWS,), jnp.float32)
    for _ in range(K):
        front = slabs[0]
        m = jnp.min(front, axis=1)
        eq = front == m[:, None]
        n_eq = jnp.sum(jnp.where(eq, 1.0, 0.0), axis=1)
        w = jnp.clip(jnp.float32(K) - cnt, 0.0, n_eq)
        mbit = jax.lax.bitcast_convert_type(m, jnp.uint32) & jnp.uint32(1)
        is_same = mbit.astype(jnp.float32)
        contrib = jnp.where(w > 0.0, w * m, 0.0)
        num = num + contrib * (1.0 - is_same)
        den = den + contrib * is_same
        cnt = cnt + n_eq
        for s in range(SLABS - 1):
            slabs[s] = jnp.where(eq, slabs[s + 1], slabs[s])
        slabs[SLABS - 1] = jnp.where(eq, jnp.float32(MASKED),
                                     slabs[SLABS - 1])

    out_ref[0, 0, 0, :] = num / (den + 1e-8)


@functools.partial(jax.jit, static_argnames=("interpret",))
def kernel(train_feature, train_label, interpret=False):
    B, N, C = train_feature.shape
    L = train_label.shape[-1]
    sq = jnp.sum(train_feature * train_feature, axis=-1)
    sq3 = sq[:, None, :]
    grid = (B, N // ROWS)
    return pl.pallas_call(
        _knn_body,
        grid=grid,
        in_specs=[
            pl.BlockSpec((1, ROWS, C), lambda b, i: (b, i, 0)),
            pl.BlockSpec((1, N, C), lambda b, i: (b, 0, 0)),
            pl.BlockSpec((1, ROWS, L), lambda b, i: (b, i, 0)),
            pl.BlockSpec((1, N, L), lambda b, i: (b, 0, 0)),
            pl.BlockSpec((1, 1, ROWS), lambda b, i: (b, 0, i)),
            pl.BlockSpec((1, 1, N), lambda b, i: (b, 0, 0)),
        ],
        out_specs=pl.BlockSpec((1, 1, 1, ROWS), lambda b, i: (b, i, 0, 0)),
        out_shape=jax.ShapeDtypeStruct((B, N // ROWS, 1, ROWS), jnp.float32),
        interpret=interpret,
    )(train_feature, train_feature, train_label, train_label, sq3, sq3
      ).reshape(B, N)


# speedup vs baseline: 39.2642x; 1.4752x over previous
import functools

import jax
import jax.numpy as jnp
from jax.experimental import pallas as pl
from jax.experimental.pallas import tpu as pltpu

K = 16
QCOLS = 256
MASKED = 1e30
SLABS = 16


def _oddeven_merge(lo, hi, r, pairs):
    step = r * 2
    if step < hi - lo:
        _oddeven_merge(lo, hi, step, pairs)
        _oddeven_merge(lo + r, hi, step, pairs)
        for i in range(lo + r, hi - r, step):
            pairs.append((i, i + r))
    else:
        pairs.append((lo, lo + r))


def _oddeven_merge_sort(lo, hi, pairs):
    if hi - lo >= 1:
        mid = lo + (hi - lo) // 2
        _oddeven_merge_sort(lo, mid, pairs)
        _oddeven_merge_sort(mid + 1, hi, pairs)
        _oddeven_merge(lo, hi, 1, pairs)


_SORT_PAIRS = []
_oddeven_merge_sort(0, SLABS - 1, _SORT_PAIRS)


def _knn_body(src_ref, all_ref, lab_ref, alllab_ref, sqs_ref, sqd_ref,
              out_ref):
    src = src_ref[0]
    allf = all_ref[0]
    lab = lab_ref[0]
    alllab = alllab_ref[0]

    dn = (((1,), (1,)), ((), ()))
    dots = jax.lax.dot_general(allf, src, dn)
    same = jax.lax.dot_general(alllab, lab, dn,
                               preferred_element_type=jnp.int32)

    sq_s = sqs_ref[0, 0]
    sq_d = sqd_ref[0, 0]
    dist = sq_s[None, :] - 2.0 * dots + sq_d[:, None]
    dist = jnp.maximum(dist, 0.0)

    bits = jax.lax.bitcast_convert_type(dist, jnp.uint32)
    keys = jax.lax.bitcast_convert_type(
        (bits & jnp.uint32(0xFFFFFFFE))
        | jax.lax.bitcast_convert_type(same, jnp.uint32),
        jnp.float32)

    N = keys.shape[0]
    W = N // SLABS
    slabs = [keys[s * W:(s + 1) * W, :] for s in range(SLABS)]
    for i, j in _SORT_PAIRS:
        a, b = slabs[i], slabs[j]
        slabs[i] = jnp.minimum(a, b)
        slabs[j] = jnp.maximum(a, b)

    num = jnp.zeros((QCOLS,), jnp.float32)
    den = jnp.zeros((QCOLS,), jnp.float32)
    cnt = jnp.zeros((QCOLS,), jnp.float32)
    for t in range(K):
        front = slabs[0]
        m = jnp.min(front, axis=0)
        eq = front == m[None, :]
        n_eq = jnp.sum(jnp.where(eq, 1.0, 0.0), axis=0)
        w = jnp.clip(jnp.float32(K) - cnt, 0.0, n_eq)
        mbit = jax.lax.bitcast_convert_type(m, jnp.uint32) & jnp.uint32(1)
        is_same = mbit.astype(jnp.float32)
        contrib = jnp.where(w > 0.0, w * m, 0.0)
        num = num + contrib * (1.0 - is_same)
        den = den + contrib * is_same
        cnt = cnt + n_eq
        horizon = K - 1 - t
        for s in range(horizon):
            slabs[s] = jnp.where(eq, slabs[s + 1], slabs[s])

    out_ref[0, 0, 0, :] = num / (den + 1e-8)


@functools.partial(jax.jit, static_argnames=("interpret",))
def kernel(train_feature, train_label, interpret=False):
    B, N, C = train_feature.shape
    L = train_label.shape[-1]
    sq = jnp.sum(train_feature * train_feature, axis=-1)
    sq3 = sq[:, None, :]
    lab8 = train_label.astype(jnp.int8)
    grid = (B, N // QCOLS)
    return pl.pallas_call(
        _knn_body,
        grid=grid,
        in_specs=[
            pl.BlockSpec((1, QCOLS, C), lambda b, i: (b, i, 0)),
            pl.BlockSpec((1, N, C), lambda b, i: (b, 0, 0)),
            pl.BlockSpec((1, QCOLS, L), lambda b, i: (b, i, 0)),
            pl.BlockSpec((1, N, L), lambda b, i: (b, 0, 0)),
            pl.BlockSpec((1, 1, QCOLS), lambda b, i: (b, 0, i)),
            pl.BlockSpec((1, 1, N), lambda b, i: (b, 0, 0)),
        ],
        out_specs=pl.BlockSpec((1, 1, 1, QCOLS), lambda b, i: (b, i, 0, 0)),
        out_shape=jax.ShapeDtypeStruct((B, N // QCOLS, 1, QCOLS),
                                       jnp.float32),
        interpret=interpret,
    )(train_feature, train_feature, lab8, lab8, sq3, sq3).reshape(B, N)


# speedup vs baseline: 60.6859x; 1.5456x over previous
import functools

import jax
import jax.numpy as jnp
from jax.experimental import pallas as pl
from jax.experimental.pallas import tpu as pltpu

K = 16
QCOLS = 256
MASKED = 1e30
SLABS = 16


def _oddeven_merge(lo, hi, r, pairs):
    step = r * 2
    if step < hi - lo:
        _oddeven_merge(lo, hi, step, pairs)
        _oddeven_merge(lo + r, hi, step, pairs)
        for i in range(lo + r, hi - r, step):
            pairs.append((i, i + r))
    else:
        pairs.append((lo, lo + r))


def _oddeven_merge_sort(lo, hi, pairs):
    if hi - lo >= 1:
        mid = lo + (hi - lo) // 2
        _oddeven_merge_sort(lo, mid, pairs)
        _oddeven_merge_sort(mid + 1, hi, pairs)
        _oddeven_merge(lo, hi, 1, pairs)


_SORT_PAIRS = []
_oddeven_merge_sort(0, SLABS - 1, _SORT_PAIRS)


def _knn_body(src_ref, all_ref, lab_ref, alllab_ref, sqs_ref, sqd_ref,
              out_ref):
    src = src_ref[0]
    allf = all_ref[0]
    lab = lab_ref[0]
    alllab = alllab_ref[0]

    dn = (((1,), (1,)), ((), ()))
    dots = jax.lax.dot_general(allf, src, dn)
    same = jax.lax.dot_general(alllab, lab, dn,
                               preferred_element_type=jnp.int32)

    sq_s = sqs_ref[0, 0]
    sq_d = sqd_ref[0, 0]
    dist = sq_s[None, :] - 2.0 * dots + sq_d[:, None]
    dist = jnp.maximum(dist, 0.0)

    bits = jax.lax.bitcast_convert_type(dist, jnp.uint32)
    keys = jax.lax.bitcast_convert_type(
        (bits & jnp.uint32(0xFFFFFFFE))
        | jax.lax.bitcast_convert_type(same, jnp.uint32),
        jnp.float32)

    N = keys.shape[0]
    W = N // SLABS
    slabs = [keys[s * W:(s + 1) * W, :] for s in range(SLABS)]
    for i, j in _SORT_PAIRS:
        a, b = slabs[i], slabs[j]
        slabs[i] = jnp.minimum(a, b)
        slabs[j] = jnp.maximum(a, b)

    w = W
    while w > 1:
        h = w // 2
        a = [s[:h, :] for s in slabs]
        b = [s[h:, :] for s in slabs]
        slabs = [jnp.minimum(a[s], b[SLABS - 1 - s]) for s in range(SLABS)]
        for stride in (8, 4, 2, 1):
            for i in range(SLABS):
                if i & stride == 0:
                    lo = jnp.minimum(slabs[i], slabs[i + stride])
                    hi = jnp.maximum(slabs[i], slabs[i + stride])
                    slabs[i], slabs[i + stride] = lo, hi
        w = h

    num = jnp.zeros((QCOLS,), jnp.float32)
    den = jnp.zeros((QCOLS,), jnp.float32)
    for s in range(SLABS):
        v = slabs[s][0]
        vbit = jax.lax.bitcast_convert_type(v, jnp.uint32) & jnp.uint32(1)
        is_same = vbit.astype(jnp.float32)
        num = num + v * (1.0 - is_same)
        den = den + v * is_same

    out_ref[0, 0, 0, :] = num / (den + 1e-8)


@functools.partial(jax.jit, static_argnames=("interpret",))
def kernel(train_feature, train_label, interpret=False):
    B, N, C = train_feature.shape
    L = train_label.shape[-1]
    sq = jnp.sum(train_feature * train_feature, axis=-1)
    sq3 = sq[:, None, :]
    lab8 = train_label.astype(jnp.int8)
    grid = (B, N // QCOLS)
    return pl.pallas_call(
        _knn_body,
        grid=grid,
        in_specs=[
            pl.BlockSpec((1, QCOLS, C), lambda b, i: (b, i, 0)),
            pl.BlockSpec((1, N, C), lambda b, i: (b, 0, 0)),
            pl.BlockSpec((1, QCOLS, L), lambda b, i: (b, i, 0)),
            pl.BlockSpec((1, N, L), lambda b, i: (b, 0, 0)),
            pl.BlockSpec((1, 1, QCOLS), lambda b, i: (b, 0, i)),
            pl.BlockSpec((1, 1, N), lambda b, i: (b, 0, 0)),
        ],
        out_specs=pl.BlockSpec((1, 1, 1, QCOLS), lambda b, i: (b, i, 0, 0)),
        out_shape=jax.ShapeDtypeStruct((B, N // QCOLS, 1, QCOLS),
                                       jnp.float32),
        interpret=interpret,
    )(train_feature, train_feature, lab8, lab8, sq3, sq3).reshape(B, N)


# speedup vs baseline: 62.5100x; 1.0301x over previous
import functools

import jax
import jax.numpy as jnp
from jax.experimental import pallas as pl
from jax.experimental.pallas import tpu as pltpu

K = 16
QCOLS = 256
MASKED = 1e30
SLABS = 16


def _oddeven_merge(lo, hi, r, pairs):
    step = r * 2
    if step < hi - lo:
        _oddeven_merge(lo, hi, step, pairs)
        _oddeven_merge(lo + r, hi, step, pairs)
        for i in range(lo + r, hi - r, step):
            pairs.append((i, i + r))
    else:
        pairs.append((lo, lo + r))


def _oddeven_merge_sort(lo, hi, pairs):
    if hi - lo >= 1:
        mid = lo + (hi - lo) // 2
        _oddeven_merge_sort(lo, mid, pairs)
        _oddeven_merge_sort(mid + 1, hi, pairs)
        _oddeven_merge(lo, hi, 1, pairs)


_SORT_PAIRS = []
_oddeven_merge_sort(0, SLABS - 1, _SORT_PAIRS)


def _knn_body(src_ref, all_ref, lab_ref, alllab_ref, sqs_ref, sqd_ref,
              out_ref):
    src = src_ref[0]
    allf = all_ref[0]
    lab = lab_ref[0]
    alllab = alllab_ref[0]

    dn = (((1,), (1,)), ((), ()))
    dots = jax.lax.dot_general(allf, src, dn)
    same = jax.lax.dot_general(alllab, lab, dn,
                               preferred_element_type=jnp.int32)

    sq_s = sqs_ref[0, 0]
    sq_d = sqd_ref[0, 0]
    dist = sq_s[None, :] - 2.0 * dots + sq_d[:, None]

    bits = jax.lax.bitcast_convert_type(dist, jnp.uint32)
    keys = jax.lax.bitcast_convert_type(
        (bits & jnp.uint32(0xFFFFFFFE))
        | jax.lax.bitcast_convert_type(same, jnp.uint32),
        jnp.float32)

    N = keys.shape[0]
    W = N // SLABS
    slabs = [keys[s * W:(s + 1) * W, :] for s in range(SLABS)]
    for i, j in _SORT_PAIRS:
        a, b = slabs[i], slabs[j]
        slabs[i] = jnp.minimum(a, b)
        slabs[j] = jnp.maximum(a, b)

    w = W
    while w > 1:
        h = w // 2
        a = [s[:h, :] for s in slabs]
        b = [s[h:, :] for s in slabs]
        slabs = [jnp.minimum(a[s], b[SLABS - 1 - s]) for s in range(SLABS)]
        for stride in (8, 4, 2, 1):
            for i in range(SLABS):
                if i & stride == 0:
                    lo = jnp.minimum(slabs[i], slabs[i + stride])
                    hi = jnp.maximum(slabs[i], slabs[i + stride])
                    slabs[i], slabs[i + stride] = lo, hi
        w = h

    num = jnp.zeros((QCOLS,), jnp.float32)
    den = jnp.zeros((QCOLS,), jnp.float32)
    for s in range(SLABS):
        v = slabs[s][0]
        vbit = jax.lax.bitcast_convert_type(v, jnp.uint32) & jnp.uint32(1)
        is_same = vbit.astype(jnp.float32)
        v = jnp.maximum(v, 0.0)
        num = num + v * (1.0 - is_same)
        den = den + v * is_same

    out_ref[0, 0, 0, :] = num / (den + 1e-8)


@functools.partial(jax.jit, static_argnames=("interpret",))
def kernel(train_feature, train_label, interpret=False):
    B, N, C = train_feature.shape
    L = train_label.shape[-1]
    sq = jnp.sum(train_feature * train_feature, axis=-1)
    sq3 = sq[:, None, :]
    lab8 = train_label.astype(jnp.int8)
    grid = (B, N // QCOLS)
    return pl.pallas_call(
        _knn_body,
        grid=grid,
        in_specs=[
            pl.BlockSpec((1, QCOLS, C), lambda b, i: (b, i, 0)),
            pl.BlockSpec((1, N, C), lambda b, i: (b, 0, 0)),
            pl.BlockSpec((1, QCOLS, L), lambda b, i: (b, i, 0)),
            pl.BlockSpec((1, N, L), lambda b, i: (b, 0, 0)),
            pl.BlockSpec((1, 1, QCOLS), lambda b, i: (b, 0, i)),
            pl.BlockSpec((1, 1, N), lambda b, i: (b, 0, 0)),
        ],
        out_specs=pl.BlockSpec((1, 1, 1, QCOLS), lambda b, i: (b, i, 0, 0)),
        out_shape=jax.ShapeDtypeStruct((B, N // QCOLS, 1, QCOLS),
                                       jnp.float32),
        interpret=interpret,
    )(train_feature, train_feature, lab8, lab8, sq3, sq3).reshape(B, N)
